# Initial kernel scaffold; baseline (speedup 1.0000x reference)
import jax
import jax.numpy as jnp
from jax.experimental import pallas as pl


def kernel(x0, x1, x2, x3, x4, Wc, bc, Wr, br, Wo, bo):
    raise NotImplementedError("write your pallas kernel here")


# speedup vs baseline: 9.2031x; 9.2031x over previous
import functools
import math

import jax
import jax.numpy as jnp
from jax import lax
from jax.experimental import pallas as pl
from jax.experimental.pallas import tpu as pltpu
from jax.experimental.pallas import tpu_sc as plsc

NUM_ANCHORS = 3
NMS_THR = 0.8
TOPK = 2000


def _next_pow2(n):
    p = 1
    while p < n:
        p *= 2
    return p


def _gather_rows(tabf, idxf):
    G = idxf.shape[0]
    nw = 32
    rpw = G // nw
    mesh = plsc.VectorSubcoreMesh(core_axis_name="c", subcore_axis_name="s")

    @functools.partial(
        pl.kernel, mesh=mesh,
        out_type=jax.ShapeDtypeStruct((G, 8), jnp.float32),
        scratch_types=[
            pltpu.VMEM((rpw,), jnp.int32),
            pltpu.VMEM((rpw, 8), jnp.float32),
            pltpu.SemaphoreType.DMA,
        ],
        compiler_params=pltpu.CompilerParams(use_tc_tiling_on_sc=False),
    )
    def gk(tab_hbm, idx_hbm, out_hbm, idx_v, rows_v, sem):
        wid = lax.axis_index("s") * 2 + lax.axis_index("c")
        base = wid * rpw
        pltpu.sync_copy(idx_hbm.at[pl.ds(base, rpw)], idx_v)
        pltpu.async_copy(tab_hbm.at[idx_v], rows_v, sem).wait()
        pltpu.sync_copy(rows_v, out_hbm.at[pl.ds(base, rpw)])

    return gk(tabf, idxf)


def _nms_body(K, Kp, vt_ref, val_ref, out_ref, msk_ref):
    Sk = Kp // 128
    f32 = jnp.float32
    comps = [vt_ref[0, c] for c in range(8)]
    v1x, v1y, v2x, v2y, v3x, v3y, v4x, v4y = comps
    x1 = jnp.minimum(jnp.minimum(v1x, v2x), jnp.minimum(v3x, v4x))
    y1 = jnp.minimum(jnp.minimum(v1y, v2y), jnp.minimum(v3y, v4y))
    x2 = jnp.maximum(jnp.maximum(v1x, v2x), jnp.maximum(v3x, v4x))
    y2 = jnp.maximum(jnp.maximum(v1y, v2y), jnp.maximum(v3y, v4y))
    area = jnp.maximum(x2 - x1, 0.0) * jnp.maximum(y2 - y1, 0.0)

    jg = (lax.broadcasted_iota(jnp.int32, (Sk, 128), 0) * 128
          + lax.broadcasted_iota(jnp.int32, (Sk, 128), 1))
    valid = jg < K
    lane = lax.broadcasted_iota(jnp.int32, (128,), 0)
    keep = valid.astype(f32)

    nblk = (K + 127) // 128
    for bb in range(nblk):
        rows_bb = min(128, K - bb * 128)
        def col(v):
            return jnp.transpose(v[bb:bb + 1], (1, 0))[:, :, None]
        ix1 = jnp.maximum(col(x1), x1[None])
        iy1 = jnp.maximum(col(y1), y1[None])
        ix2 = jnp.minimum(col(x2), x2[None])
        iy2 = jnp.minimum(col(y2), y2[None])
        inter = (jnp.maximum(ix2 - ix1, 0.0)
                 * jnp.maximum(iy2 - iy1, 0.0))
        iou = inter / (col(area) + area[None] - inter + 1e-9)
        ig = bb * 128 + lax.broadcasted_iota(jnp.int32, (128, 1, 1), 0)
        sup = ((iou > NMS_THR) & (jg[None] > ig)
               & valid[None]).astype(f32)
        msk_ref[...] = sup

        def body(r, keep):
            ki = jnp.sum(jnp.where(lane == r, keep[bb], 0.0))
            row = msk_ref[r]
            return keep * (1.0 - ki * row)

        keep = lax.fori_loop(0, rows_bb, body, keep)

    outs = comps + [val_ref[0]]
    for ci, comp in enumerate(outs):
        out_ref[0, ci] = comp * keep
    zero = jnp.zeros((Sk, 128), f32)
    for ci in range(9, 16):
        out_ref[0, ci] = zero


def _nms(vt, valst, K, Kp):
    B = vt.shape[0]
    Sk = Kp // 128
    body = functools.partial(_nms_body, K, Kp)
    return pl.pallas_call(
        body,
        grid=(B,),
        in_specs=[
            pl.BlockSpec((1, 8, Sk, 128), lambda b: (b, 0, 0, 0)),
            pl.BlockSpec((1, Sk, 128), lambda b: (b, 0, 0)),
        ],
        out_specs=pl.BlockSpec((1, 16, Sk, 128), lambda b: (b, 0, 0, 0)),
        out_shape=jax.ShapeDtypeStruct((B, 16, Sk, 128), jnp.float32),
        scratch_shapes=[pltpu.VMEM((128, Sk, 128), jnp.float32)],
    )(vt, valst)


def _anchors(h, w):
    ratios = jnp.array([0.5, 1.0, 2.0], dtype=jnp.float32)
    base = 8.0
    aw = base * jnp.sqrt(ratios)
    ah = base / jnp.sqrt(ratios)
    ys = jnp.arange(h, dtype=jnp.float32) + 0.5
    xs = jnp.arange(w, dtype=jnp.float32) + 0.5
    cx = jnp.broadcast_to(xs[None, None, :], (NUM_ANCHORS, h, w))
    cy = jnp.broadcast_to(ys[None, :, None], (NUM_ANCHORS, h, w))
    aws = jnp.broadcast_to(aw[:, None, None], (NUM_ANCHORS, h, w))
    ahs = jnp.broadcast_to(ah[:, None, None], (NUM_ANCHORS, h, w))
    return jnp.stack([cx, cy, aws, ahs], -1).reshape(-1, 4)


def _decode(reg, anchors):
    ax, ay, aw, ah = anchors[:, 0], anchors[:, 1], anchors[:, 2], anchors[:, 3]
    dx, dy, dw, dh, da, db = [reg[..., i] for i in range(6)]
    w = aw[None, :] * jnp.exp(jnp.clip(dw, -4.0, 4.0))
    h = ah[None, :] * jnp.exp(jnp.clip(dh, -4.0, 4.0))
    x = ax[None, :] + dx * aw[None, :]
    y = ay[None, :] + dy * ah[None, :]
    da_ = da * w
    db_ = db * h
    v1 = jnp.stack([x + da_, y - h / 2], -1)
    v2 = jnp.stack([x + w / 2, y + db_], -1)
    v3 = jnp.stack([x - da_, y + h / 2], -1)
    v4 = jnp.stack([x - w / 2, y - db_], -1)
    return jnp.stack([v1, v2, v3, v4], 2)


def kernel(x0, x1, x2, x3, x4, Wc, bc, Wr, br, Wo, bo):
    B = x0.shape[0]
    outs = []
    for x in (x0, x1, x2, x3, x4):
        H, W = x.shape[2], x.shape[3]
        N = NUM_ANCHORS * H * W
        K = min(TOPK, N)
        Kp = 128 * ((K + 127) // 128)
        if (B * Kp // 32) % 8:
            Kp = 256 * ((K + 255) // 256)
        Sk = Kp // 128
        z = lax.conv_general_dilated(
            x, Wc, (1, 1), 'SAME',
            dimension_numbers=('NCHW', 'OIHW', 'NCHW')) + bc[None, :, None, None]
        z = jax.nn.relu(z)
        reg = (jnp.einsum('bchw,oc->bohw', z, Wr) + br[None, :, None, None])
        obj = (jnp.einsum('bchw,oc->bohw', z, Wo) + bo[None, :, None, None])
        reg = reg.reshape(B, NUM_ANCHORS, 6, H, W).transpose(
            0, 1, 3, 4, 2).reshape(B, -1, 6)
        obj = obj.reshape(B, -1)
        verts = _decode(reg, _anchors(H, W))
        scores = jax.nn.sigmoid(obj)
        vals, idx = lax.top_k(scores, K)

        vflat = verts.reshape(B * N, 8)
        idxp = jnp.concatenate(
            [idx, jnp.zeros((B, Kp - K), jnp.int32)], axis=1)
        idxf = (idxp + (jnp.arange(B, dtype=jnp.int32) * N)[:, None]
                ).reshape(B * Kp)
        rows = _gather_rows(vflat, idxf)
        vt = rows.reshape(B, Kp, 8).transpose(0, 2, 1).reshape(B, 8, Sk, 128)
        valsp = jnp.concatenate(
            [vals, jnp.zeros((B, Kp - K), jnp.float32)],
            axis=1).reshape(B, Sk, 128)
        outt = _nms(vt, valsp, K, Kp)
        out_l = outt.reshape(B, 16, Kp).transpose(0, 2, 1)[:, :K, :9]
        outs.append(out_l)
    return jnp.concatenate(outs, axis=1)
